# Initial kernel scaffold; baseline (speedup 1.0000x reference)
import functools

import jax
import jax.numpy as jnp
from jax import lax
from jax.experimental import pallas as pl
from jax.experimental.pallas import tpu as pltpu
from jax.experimental.pallas import tpu_sc as plsc

_NC = 2
_NS = 16
_NW = _NC * _NS
_CH = 128


def _sc_segment_sum(n_nodes: int, n_edges: int, de: int):
    assert n_edges % _CH == 0
    n_chunks = n_edges // _CH
    base, rem = divmod(n_chunks, _NW)
    assert n_nodes % _NS == 0
    rows_per_sub = n_nodes // _NS

    mesh = plsc.VectorSubcoreMesh(core_axis_name="c", subcore_axis_name="s")

    @functools.partial(
        pl.kernel,
        out_type=jax.ShapeDtypeStruct((_NC, n_nodes, de), jnp.float32),
        mesh=mesh,
        scratch_types=[
            pltpu.VMEM((1, _CH), jnp.int32),
            pltpu.VMEM((1, _CH, de), jnp.float32),
            pltpu.VMEM_SHARED((n_nodes, de), jnp.float32),
        ],
    )
    def scatter_kernel(col_hbm, attr_hbm, zero_hbm, out_hbm, idx_v, rows_v, acc_sh):
        c = lax.axis_index("c")
        s = lax.axis_index("s")
        wid = s * _NC + c

        zoff = s * rows_per_sub
        pltpu.sync_copy(
            zero_hbm.at[pl.ds(zoff, rows_per_sub), :],
            acc_sh.at[pl.ds(zoff, rows_per_sub), :],
        )
        plsc.subcore_barrier()

        start = wid * base + jnp.minimum(wid, rem)
        count = base + (wid < rem).astype(jnp.int32)

        def chunk_body(j, carry):
            ebase = (start + j) * _CH
            pltpu.sync_copy(col_hbm.at[pl.ds(ebase, _CH)], idx_v.at[0])
            pltpu.sync_copy(attr_hbm.at[pl.ds(ebase, _CH), :], rows_v.at[0])
            pltpu.sync_copy(rows_v.at[0], acc_sh.at[idx_v.at[0]], add=True)
            return carry

        lax.fori_loop(0, count, chunk_body, 0)
        plsc.subcore_barrier()

        pltpu.sync_copy(
            acc_sh.at[pl.ds(zoff, rows_per_sub), :],
            out_hbm.at[c, pl.ds(zoff, rows_per_sub), :],
        )

    return scatter_kernel


def _mlp(n_nodes: int, d: int, de: int, g: int, gf: int, h: int, block: int):
    assert n_nodes % block == 0
    grid = (n_nodes // block,)

    def body(x_ref, pp_ref, batch_ref, u_ref, w1_ref, b1_ref, w2_ref, b2_ref, out_ref):
        x_b = x_ref[...]
        agg = pp_ref[0] + pp_ref[1]
        bcol = batch_ref[...]
        onehot = (bcol == lax.broadcasted_iota(jnp.int32, (block, g), 1)).astype(
            jnp.float32
        )
        ub = jnp.dot(onehot, u_ref[...], preferred_element_type=jnp.float32)
        hcat = jnp.concatenate([x_b, agg, ub], axis=1)
        hid = jnp.dot(hcat, w1_ref[...], preferred_element_type=jnp.float32)
        hid = jnp.maximum(hid + b1_ref[...], 0.0)
        out = jnp.dot(hid, w2_ref[...], preferred_element_type=jnp.float32)
        out_ref[...] = out + b2_ref[...] + x_b

    return pl.pallas_call(
        body,
        grid=grid,
        in_specs=[
            pl.BlockSpec((block, d), lambda i: (i, 0)),
            pl.BlockSpec((_NC, block, de), lambda i: (0, i, 0)),
            pl.BlockSpec((block, 1), lambda i: (i, 0)),
            pl.BlockSpec((g, gf), lambda i: (0, 0)),
            pl.BlockSpec((d + de + gf, h), lambda i: (0, 0)),
            pl.BlockSpec((1, h), lambda i: (0, 0)),
            pl.BlockSpec((h, d), lambda i: (0, 0)),
            pl.BlockSpec((1, d), lambda i: (0, 0)),
        ],
        out_specs=pl.BlockSpec((block, d), lambda i: (i, 0)),
        out_shape=jax.ShapeDtypeStruct((n_nodes, d), jnp.float32),
        compiler_params=pltpu.CompilerParams(
            dimension_semantics=("parallel",),
        ),
    )


def kernel(x, edge_index, edge_attr, u, batch, W1, b1, W2, b2):
    n_nodes, d = x.shape
    n_edges, de = edge_attr.shape
    g, gf = u.shape
    h = W1.shape[1]

    col = edge_index[1]
    zeros = jnp.zeros((n_nodes, de), jnp.float32)
    partials = _sc_segment_sum(n_nodes, n_edges, de)(col, edge_attr, zeros)

    mlp = _mlp(n_nodes, d, de, g, gf, h, block=1000)
    return mlp(
        x,
        partials,
        batch.reshape(n_nodes, 1),
        u,
        W1,
        b1.reshape(1, h),
        W2,
        b2.reshape(1, d),
    )


# speedup vs baseline: 3.7121x; 3.7121x over previous
import functools

import jax
import jax.numpy as jnp
from jax import lax
from jax.experimental import pallas as pl
from jax.experimental.pallas import tpu as pltpu
from jax.experimental.pallas import tpu_sc as plsc

_NC = 2
_NS = 16
_NW = _NC * _NS
_CH = 128


def _sc_segment_sum(n_nodes: int, n_edges: int, de: int):
    assert n_edges % _CH == 0
    n_chunks = n_edges // _CH
    base, rem = divmod(n_chunks, _NW)
    rows_per_sub = (n_nodes // _NS) // 8 * 8
    tail_rows = n_nodes - rows_per_sub * _NS

    mesh = plsc.VectorSubcoreMesh(core_axis_name="c", subcore_axis_name="s")

    @functools.partial(
        pl.kernel,
        out_type=jax.ShapeDtypeStruct((_NC, n_nodes, de), jnp.float32),
        mesh=mesh,
        scratch_types=[
            pltpu.VMEM((_CH,), jnp.int32),
            pltpu.VMEM((_CH, de), jnp.float32),
            pltpu.VMEM_SHARED((n_nodes, de), jnp.float32),
        ],
        compiler_params=pltpu.CompilerParams(use_tc_tiling_on_sc=False),
    )
    def scatter_kernel(col_hbm, attr_hbm, zero_hbm, out_hbm, idx_v, rows_v, acc_sh):
        c = lax.axis_index("c")
        s = lax.axis_index("s")
        wid = s * _NC + c

        zoff = s * rows_per_sub
        pltpu.sync_copy(
            zero_hbm.at[pl.ds(zoff, rows_per_sub), :],
            acc_sh.at[pl.ds(zoff, rows_per_sub), :],
        )
        if tail_rows:
            toff = rows_per_sub * _NS

            @pl.when(s == _NS - 1)
            def _zero_tail():
                pltpu.sync_copy(
                    zero_hbm.at[pl.ds(toff, tail_rows), :],
                    acc_sh.at[pl.ds(toff, tail_rows), :],
                )

        plsc.subcore_barrier()

        start = wid * base + jnp.minimum(wid, rem)
        count = base + (wid < rem).astype(jnp.int32)

        def chunk_body(j, carry):
            ebase = (start + j) * _CH
            pltpu.sync_copy(col_hbm.at[pl.ds(ebase, _CH)], idx_v)
            pltpu.sync_copy(attr_hbm.at[pl.ds(ebase, _CH), :], rows_v)
            pltpu.sync_copy(rows_v, acc_sh.at[idx_v], add=True)
            return carry

        lax.fori_loop(0, count, chunk_body, 0)
        plsc.subcore_barrier()

        pltpu.sync_copy(
            acc_sh.at[pl.ds(zoff, rows_per_sub), :],
            out_hbm.at[c, pl.ds(zoff, rows_per_sub), :],
        )
        if tail_rows:
            toff2 = rows_per_sub * _NS

            @pl.when(s == _NS - 1)
            def _write_tail():
                pltpu.sync_copy(
                    acc_sh.at[pl.ds(toff2, tail_rows), :],
                    out_hbm.at[c, pl.ds(toff2, tail_rows), :],
                )

    return scatter_kernel


def _mlp(n_nodes: int, d: int, de: int, g: int, gf: int, h: int, block: int):
    assert n_nodes % block == 0
    grid = (n_nodes // block,)

    def body(x_ref, pp_ref, batch_ref, u_ref, w1_ref, b1_ref, w2_ref, b2_ref, out_ref):
        x_b = x_ref[...]
        agg = pp_ref[0] + pp_ref[1]
        bcol = batch_ref[...]
        onehot = (bcol == lax.broadcasted_iota(jnp.int32, (block, g), 1)).astype(
            jnp.float32
        )
        ub = jnp.dot(onehot, u_ref[...], preferred_element_type=jnp.float32)
        hcat = jnp.concatenate([x_b, agg, ub], axis=1)
        hid = jnp.dot(hcat, w1_ref[...], preferred_element_type=jnp.float32)
        hid = jnp.maximum(hid + b1_ref[...], 0.0)
        out = jnp.dot(hid, w2_ref[...], preferred_element_type=jnp.float32)
        out_ref[...] = out + b2_ref[...] + x_b

    return pl.pallas_call(
        body,
        grid=grid,
        in_specs=[
            pl.BlockSpec((block, d), lambda i: (i, 0)),
            pl.BlockSpec((_NC, block, de), lambda i: (0, i, 0)),
            pl.BlockSpec((block, 1), lambda i: (i, 0)),
            pl.BlockSpec((g, gf), lambda i: (0, 0)),
            pl.BlockSpec((d + de + gf, h), lambda i: (0, 0)),
            pl.BlockSpec((1, h), lambda i: (0, 0)),
            pl.BlockSpec((h, d), lambda i: (0, 0)),
            pl.BlockSpec((1, d), lambda i: (0, 0)),
        ],
        out_specs=pl.BlockSpec((block, d), lambda i: (i, 0)),
        out_shape=jax.ShapeDtypeStruct((n_nodes, d), jnp.float32),
        compiler_params=pltpu.CompilerParams(
            dimension_semantics=("parallel",),
        ),
    )


def kernel(x, edge_index, edge_attr, u, batch, W1, b1, W2, b2):
    n_nodes, d = x.shape
    n_edges, de = edge_attr.shape
    g, gf = u.shape
    h = W1.shape[1]

    col = edge_index[1]
    zeros = jnp.zeros((n_nodes, de), jnp.float32)
    partials = _sc_segment_sum(n_nodes, n_edges, de)(col, edge_attr, zeros)

    mlp = _mlp(n_nodes, d, de, g, gf, h, block=1000)
    return mlp(
        x,
        partials,
        batch.reshape(n_nodes, 1),
        u,
        W1,
        b1.reshape(1, h),
        W2,
        b2.reshape(1, d),
    )


# speedup vs baseline: 5.4064x; 1.4564x over previous
import functools

import jax
import jax.numpy as jnp
from jax import lax
from jax.experimental import pallas as pl
from jax.experimental.pallas import tpu as pltpu
from jax.experimental.pallas import tpu_sc as plsc

_NC = 2
_NS = 16
_NW = _NC * _NS
_CH = 128


def _sc_segment_sum(n_nodes: int, n_edges: int, de: int):
    assert n_edges % _CH == 0
    n_chunks = n_edges // _CH
    base, rem = divmod(n_chunks, _NW)
    ss = 13
    assert base % ss == 0
    n_ss = base // ss
    rows_per_sub = (n_nodes // _NS) // 8 * 8
    tail_rows = n_nodes - rows_per_sub * _NS

    mesh = plsc.VectorSubcoreMesh(core_axis_name="c", subcore_axis_name="s")

    @functools.partial(
        pl.kernel,
        out_type=jax.ShapeDtypeStruct((_NC, n_nodes, de), jnp.float32),
        mesh=mesh,
        scratch_types=[
            pltpu.VMEM((2, ss, _CH), jnp.int32),
            pltpu.VMEM((2, ss, _CH, de), jnp.float32),
            pltpu.VMEM_SHARED((n_nodes, de), jnp.float32),
            pltpu.SemaphoreType.DMA,
            pltpu.SemaphoreType.DMA,
            pltpu.SemaphoreType.DMA,
        ],
        compiler_params=pltpu.CompilerParams(use_tc_tiling_on_sc=False),
    )
    def scatter_kernel(
        col_hbm, attr_hbm, zero_hbm, out_hbm, idx_v, rows_v, acc_sh,
        sem_i, sem_r, sem_s,
    ):
        c = lax.axis_index("c")
        s = lax.axis_index("s")
        wid = s * _NC + c

        zoff = s * rows_per_sub
        pltpu.sync_copy(
            zero_hbm.at[pl.ds(zoff, rows_per_sub), :],
            acc_sh.at[pl.ds(zoff, rows_per_sub), :],
        )
        if tail_rows:
            toff = rows_per_sub * _NS

            @pl.when(s == _NS - 1)
            def _zero_tail():
                pltpu.sync_copy(
                    zero_hbm.at[pl.ds(toff, tail_rows), :],
                    acc_sh.at[pl.ds(toff, tail_rows), :],
                )

        plsc.subcore_barrier()

        start = wid * base

        pending = {}

        def start_loads(t, buf):
            st = start + t * ss
            di = pltpu.async_copy(col_hbm.at[pl.ds(st, ss), :], idx_v.at[buf], sem_i)
            dr = pltpu.async_copy(attr_hbm.at[pl.ds(st, ss)], rows_v.at[buf], sem_r)
            pending[buf] = (di, dr)

        start_loads(0, 0)
        for t in range(n_ss):
            b = t % 2
            if t + 1 < n_ss:
                start_loads(t + 1, 1 - b)
            di, dr = pending[b]
            di.wait()
            dr.wait()
            scats = [
                pltpu.async_copy(
                    rows_v.at[b, j], acc_sh.at[idx_v.at[b, j]], sem_s, add=True
                )
                for j in range(ss)
            ]
            for sd in scats:
                sd.wait()

        if rem:

            @pl.when(wid < rem)
            def _extra_chunk():
                ec = _NW * base + wid
                pltpu.sync_copy(col_hbm.at[ec], idx_v.at[0, 0])
                pltpu.sync_copy(attr_hbm.at[ec], rows_v.at[0, 0])
                pltpu.sync_copy(rows_v.at[0, 0], acc_sh.at[idx_v.at[0, 0]], add=True)

        plsc.subcore_barrier()

        pltpu.sync_copy(
            acc_sh.at[pl.ds(zoff, rows_per_sub), :],
            out_hbm.at[c, pl.ds(zoff, rows_per_sub), :],
        )
        if tail_rows:
            toff2 = rows_per_sub * _NS

            @pl.when(s == _NS - 1)
            def _write_tail():
                pltpu.sync_copy(
                    acc_sh.at[pl.ds(toff2, tail_rows), :],
                    out_hbm.at[c, pl.ds(toff2, tail_rows), :],
                )

    return scatter_kernel


def _mlp(n_nodes: int, d: int, de: int, g: int, gf: int, h: int, block: int):
    assert n_nodes % block == 0
    grid = (n_nodes // block,)

    def body(x_ref, pp_ref, batch_ref, u_ref, w1_ref, b1_ref, w2_ref, b2_ref, out_ref):
        x_b = x_ref[...]
        agg = pp_ref[0] + pp_ref[1]
        bcol = batch_ref[...]
        onehot = (bcol == lax.broadcasted_iota(jnp.int32, (block, g), 1)).astype(
            jnp.float32
        )
        ub = jnp.dot(onehot, u_ref[...], preferred_element_type=jnp.float32)
        hcat = jnp.concatenate([x_b, agg, ub], axis=1)
        hid = jnp.dot(hcat, w1_ref[...], preferred_element_type=jnp.float32)
        hid = jnp.maximum(hid + b1_ref[...], 0.0)
        out = jnp.dot(hid, w2_ref[...], preferred_element_type=jnp.float32)
        out_ref[...] = out + b2_ref[...] + x_b

    return pl.pallas_call(
        body,
        grid=grid,
        in_specs=[
            pl.BlockSpec((block, d), lambda i: (i, 0)),
            pl.BlockSpec((_NC, block, de), lambda i: (0, i, 0)),
            pl.BlockSpec((block, 1), lambda i: (i, 0)),
            pl.BlockSpec((g, gf), lambda i: (0, 0)),
            pl.BlockSpec((d + de + gf, h), lambda i: (0, 0)),
            pl.BlockSpec((1, h), lambda i: (0, 0)),
            pl.BlockSpec((h, d), lambda i: (0, 0)),
            pl.BlockSpec((1, d), lambda i: (0, 0)),
        ],
        out_specs=pl.BlockSpec((block, d), lambda i: (i, 0)),
        out_shape=jax.ShapeDtypeStruct((n_nodes, d), jnp.float32),
        compiler_params=pltpu.CompilerParams(
            dimension_semantics=("parallel",),
        ),
    )


def kernel(x, edge_index, edge_attr, u, batch, W1, b1, W2, b2):
    n_nodes, d = x.shape
    n_edges, de = edge_attr.shape
    g, gf = u.shape
    h = W1.shape[1]

    col2 = edge_index[1].reshape(n_edges // _CH, _CH)
    attr3 = edge_attr.reshape(n_edges // _CH, _CH, de)
    zeros = jnp.zeros((n_nodes, de), jnp.float32)
    partials = _sc_segment_sum(n_nodes, n_edges, de)(col2, attr3, zeros)

    mlp = _mlp(n_nodes, d, de, g, gf, h, block=1000)
    return mlp(
        x,
        partials,
        batch.reshape(n_nodes, 1),
        u,
        W1,
        b1.reshape(1, h),
        W2,
        b2.reshape(1, d),
    )


# speedup vs baseline: 6.5187x; 1.2057x over previous
import functools

import jax
import jax.numpy as jnp
from jax import lax
from jax.experimental import pallas as pl
from jax.experimental.pallas import tpu as pltpu
from jax.experimental.pallas import tpu_sc as plsc

_NC = 2
_NS = 16
_NW = _NC * _NS
_CH = 128


def _sc_segment_sum(n_nodes: int, n_edges: int, de: int):
    assert n_edges % _CH == 0
    n_chunks = n_edges // _CH
    base, rem = divmod(n_chunks, _NW)
    ss = 13
    assert base % ss == 0
    n_ss = base // ss
    w = ss * _CH
    rows_per_sub = (n_nodes // _NS) // 8 * 8
    tail_rows = n_nodes - rows_per_sub * _NS

    mesh = plsc.VectorSubcoreMesh(core_axis_name="c", subcore_axis_name="s")

    @functools.partial(
        pl.kernel,
        out_type=jax.ShapeDtypeStruct((_NC, n_nodes, de), jnp.float32),
        mesh=mesh,
        scratch_types=[
            pltpu.VMEM((2, w), jnp.int32),
            pltpu.VMEM((2, de, w), jnp.float32),
            pltpu.VMEM((2, w, de), jnp.float32),
            pltpu.VMEM_SHARED((n_nodes, de), jnp.float32),
            pltpu.SemaphoreType.DMA,
            pltpu.SemaphoreType.DMA,
            pltpu.SemaphoreType.DMA,
        ],
        compiler_params=pltpu.CompilerParams(
            use_tc_tiling_on_sc=False, needs_layout_passes=False
        ),
    )
    def scatter_kernel(
        ei_hbm, attrt_hbm, zero_hbm, out_hbm, idx_v, tbuf_v, rows_v, acc_sh,
        sem_i, sem_r, sem_s,
    ):
        c = lax.axis_index("c")
        s = lax.axis_index("s")
        wid = s * _NC + c

        zoff = s * rows_per_sub
        pltpu.sync_copy(
            zero_hbm.at[pl.ds(zoff, rows_per_sub), :],
            acc_sh.at[pl.ds(zoff, rows_per_sub), :],
        )
        if tail_rows:
            toff = rows_per_sub * _NS

            @pl.when(s == _NS - 1)
            def _zero_tail():
                pltpu.sync_copy(
                    zero_hbm.at[pl.ds(toff, tail_rows), :],
                    acc_sh.at[pl.ds(toff, tail_rows), :],
                )

        plsc.subcore_barrier()

        start = wid * base
        iota16 = lax.iota(jnp.int32, 16)

        def transpose_edges(buf, n_e):
            src2d = tbuf_v.at[buf]
            dst2d = rows_v.at[buf]

            @plsc.parallel_loop(0, n_e, 1, unroll=8)
            def _tr(jj):
                vj = jnp.full((16,), jj, jnp.int32)
                row = plsc.load_gather(src2d, [iota16, vj])
                plsc.store_scatter(dst2d, [vj, iota16], row)

        pending = {}

        def start_loads(t, buf):
            st = (start + t * ss) * _CH
            descs = [pltpu.async_copy(ei_hbm.at[1, pl.ds(st, w)], idx_v.at[buf], sem_i)]
            descs += [
                pltpu.async_copy(
                    attrt_hbm.at[i, pl.ds(st, w)], tbuf_v.at[buf, i], sem_r
                )
                for i in range(de)
            ]
            pending[buf] = descs

        start_loads(0, 0)
        for t in range(n_ss):
            b = t % 2
            if t + 1 < n_ss:
                start_loads(t + 1, 1 - b)
            for dsc in pending[b]:
                dsc.wait()
            transpose_edges(b, w)
            scats = [
                pltpu.async_copy(
                    rows_v.at[b, pl.ds(j * _CH, _CH), :],
                    acc_sh.at[idx_v.at[b, pl.ds(j * _CH, _CH)]],
                    sem_s,
                    add=True,
                )
                for j in range(ss)
            ]
            for sd in scats:
                sd.wait()

        if rem:

            @pl.when(wid < rem)
            def _extra_chunk():
                ec = (_NW * base + wid) * _CH
                pltpu.sync_copy(
                    ei_hbm.at[1, pl.ds(ec, _CH)], idx_v.at[0, pl.ds(0, _CH)]
                )
                for i in range(de):
                    pltpu.sync_copy(
                        attrt_hbm.at[i, pl.ds(ec, _CH)], tbuf_v.at[0, i, pl.ds(0, _CH)]
                    )
                transpose_edges(0, _CH)
                pltpu.sync_copy(
                    rows_v.at[0, pl.ds(0, _CH), :],
                    acc_sh.at[idx_v.at[0, pl.ds(0, _CH)]],
                    add=True,
                )

        plsc.subcore_barrier()

        pltpu.sync_copy(
            acc_sh.at[pl.ds(zoff, rows_per_sub), :],
            out_hbm.at[c, pl.ds(zoff, rows_per_sub), :],
        )
        if tail_rows:
            toff2 = rows_per_sub * _NS

            @pl.when(s == _NS - 1)
            def _write_tail():
                pltpu.sync_copy(
                    acc_sh.at[pl.ds(toff2, tail_rows), :],
                    out_hbm.at[c, pl.ds(toff2, tail_rows), :],
                )

    return scatter_kernel


def _mlp(n_nodes: int, d: int, de: int, g: int, gf: int, h: int, block: int):
    assert n_nodes % block == 0
    grid = (n_nodes // block,)

    def body(x_ref, pp_ref, batch_ref, u_ref, w1_ref, b1_ref, w2_ref, b2_ref, out_ref):
        x_b = x_ref[...]
        agg = pp_ref[0] + pp_ref[1]
        bcol = batch_ref[...]
        onehot = (bcol == lax.broadcasted_iota(jnp.int32, (block, g), 1)).astype(
            jnp.float32
        )
        ub = jnp.dot(onehot, u_ref[...], preferred_element_type=jnp.float32)
        hcat = jnp.concatenate([x_b, agg, ub], axis=1)
        hid = jnp.dot(hcat, w1_ref[...], preferred_element_type=jnp.float32)
        hid = jnp.maximum(hid + b1_ref[...], 0.0)
        out = jnp.dot(hid, w2_ref[...], preferred_element_type=jnp.float32)
        out_ref[...] = out + b2_ref[...] + x_b

    return pl.pallas_call(
        body,
        grid=grid,
        in_specs=[
            pl.BlockSpec((block, d), lambda i: (i, 0)),
            pl.BlockSpec((_NC, block, de), lambda i: (0, i, 0)),
            pl.BlockSpec((block, 1), lambda i: (i, 0)),
            pl.BlockSpec((g, gf), lambda i: (0, 0)),
            pl.BlockSpec((d + de + gf, h), lambda i: (0, 0)),
            pl.BlockSpec((1, h), lambda i: (0, 0)),
            pl.BlockSpec((h, d), lambda i: (0, 0)),
            pl.BlockSpec((1, d), lambda i: (0, 0)),
        ],
        out_specs=pl.BlockSpec((block, d), lambda i: (i, 0)),
        out_shape=jax.ShapeDtypeStruct((n_nodes, d), jnp.float32),
        compiler_params=pltpu.CompilerParams(
            dimension_semantics=("parallel",),
        ),
    )


def kernel(x, edge_index, edge_attr, u, batch, W1, b1, W2, b2):
    n_nodes, d = x.shape
    n_edges, de = edge_attr.shape
    g, gf = u.shape
    h = W1.shape[1]

    zeros = jnp.zeros((n_nodes, de), jnp.float32)
    partials = _sc_segment_sum(n_nodes, n_edges, de)(edge_index, edge_attr.T, zeros)

    mlp = _mlp(n_nodes, d, de, g, gf, h, block=1000)
    return mlp(
        x,
        partials,
        batch.reshape(n_nodes, 1),
        u,
        W1,
        b1.reshape(1, h),
        W2,
        b2.reshape(1, d),
    )


# speedup vs baseline: 7.1921x; 1.1033x over previous
import functools

import jax
import jax.numpy as jnp
from jax import lax
from jax.experimental import pallas as pl
from jax.experimental.pallas import tpu as pltpu
from jax.experimental.pallas import tpu_sc as plsc

_NC = 2
_NS = 16
_NW = _NC * _NS
_CH = 128


def _sc_segment_sum(n_nodes: int, n_edges: int, de: int):
    assert n_edges % _CH == 0
    n_chunks = n_edges // _CH
    base, rem = divmod(n_chunks, _NW)
    ss = 13
    assert base % ss == 0
    n_ss = base // ss
    w = ss * _CH
    rows_per_sub = (n_nodes // _NS) // 8 * 8
    tail_rows = n_nodes - rows_per_sub * _NS

    mesh = plsc.VectorSubcoreMesh(core_axis_name="c", subcore_axis_name="s")

    @functools.partial(
        pl.kernel,
        out_type=jax.ShapeDtypeStruct((_NC, n_nodes, de), jnp.float32),
        mesh=mesh,
        scratch_types=[
            pltpu.VMEM((2, w), jnp.int32),
            pltpu.VMEM((2, de, w), jnp.float32),
            pltpu.VMEM((2, w, de), jnp.float32),
            pltpu.VMEM_SHARED((n_nodes, de), jnp.float32),
            pltpu.SemaphoreType.DMA,
            pltpu.SemaphoreType.DMA,
            pltpu.SemaphoreType.DMA,
        ],
        compiler_params=pltpu.CompilerParams(
            use_tc_tiling_on_sc=False, needs_layout_passes=False
        ),
    )
    def scatter_kernel(
        ei3_hbm, a4_hbm, zero_hbm, out_hbm, idx_v, tbuf_v, rows_v, acc_sh,
        sem_i, sem_r, sem_s,
    ):
        c = lax.axis_index("c")
        s = lax.axis_index("s")
        wid = s * _NC + c

        zoff = s * rows_per_sub
        pltpu.sync_copy(
            zero_hbm.at[pl.ds(zoff, rows_per_sub), :],
            acc_sh.at[pl.ds(zoff, rows_per_sub), :],
        )
        if tail_rows:
            toff = rows_per_sub * _NS

            @pl.when(s == _NS - 1)
            def _zero_tail():
                pltpu.sync_copy(
                    zero_hbm.at[pl.ds(toff, tail_rows), :],
                    acc_sh.at[pl.ds(toff, tail_rows), :],
                )

        plsc.subcore_barrier()

        start = wid * base
        iota16 = lax.iota(jnp.int32, 16)

        def transpose_edges(buf, n_e):
            src2d = tbuf_v.at[buf]
            dst2d = rows_v.at[buf]

            @plsc.parallel_loop(0, n_e, 1, unroll=8)
            def _tr(jj):
                vj = jnp.full((16,), jj, jnp.int32)
                row = plsc.load_gather(src2d, [iota16, vj])
                plsc.store_scatter(dst2d, [vj, iota16], row)

        pending = {}

        def start_loads(t, buf):
            st = start + t * ss
            descs = []
            for k in range(ss):
                descs.append(
                    pltpu.async_copy(
                        ei3_hbm.at[st + k, 1], idx_v.at[buf, pl.ds(k * _CH, _CH)],
                        sem_i,
                    )
                )
                for tr in range(de // 8):
                    descs.append(
                        pltpu.async_copy(
                            a4_hbm.at[tr, st + k],
                            tbuf_v.at[buf, pl.ds(tr * 8, 8), pl.ds(k * _CH, _CH)],
                            sem_r,
                        )
                    )
            pending[buf] = descs

        start_loads(0, 0)
        for t in range(n_ss):
            b = t % 2
            if t + 1 < n_ss:
                start_loads(t + 1, 1 - b)
            for dsc in pending[b]:
                dsc.wait()
            transpose_edges(b, w)
            scats = [
                pltpu.async_copy(
                    rows_v.at[b, pl.ds(j * _CH, _CH), :],
                    acc_sh.at[idx_v.at[b, pl.ds(j * _CH, _CH)]],
                    sem_s,
                    add=True,
                )
                for j in range(ss)
            ]
            for sd in scats:
                sd.wait()

        if rem:

            @pl.when(wid < rem)
            def _extra_chunk():
                ec = _NW * base + wid
                pltpu.sync_copy(ei3_hbm.at[ec, 1], idx_v.at[0, pl.ds(0, _CH)])
                for tr in range(de // 8):
                    pltpu.sync_copy(
                        a4_hbm.at[tr, ec],
                        tbuf_v.at[0, pl.ds(tr * 8, 8), pl.ds(0, _CH)],
                    )
                transpose_edges(0, _CH)
                pltpu.sync_copy(
                    rows_v.at[0, pl.ds(0, _CH), :],
                    acc_sh.at[idx_v.at[0, pl.ds(0, _CH)]],
                    add=True,
                )

        plsc.subcore_barrier()

        pltpu.sync_copy(
            acc_sh.at[pl.ds(zoff, rows_per_sub), :],
            out_hbm.at[c, pl.ds(zoff, rows_per_sub), :],
        )
        if tail_rows:
            toff2 = rows_per_sub * _NS

            @pl.when(s == _NS - 1)
            def _write_tail():
                pltpu.sync_copy(
                    acc_sh.at[pl.ds(toff2, tail_rows), :],
                    out_hbm.at[c, pl.ds(toff2, tail_rows), :],
                )

    return scatter_kernel


def _mlp(n_nodes: int, d: int, de: int, g: int, gf: int, h: int, block: int):
    assert n_nodes % block == 0
    grid = (n_nodes // block,)

    def body(x_ref, pp_ref, batch_ref, u_ref, w1_ref, b1_ref, w2_ref, b2_ref, out_ref):
        x_b = x_ref[...]
        agg = pp_ref[0] + pp_ref[1]
        bcol = batch_ref[...]
        onehot = (bcol == lax.broadcasted_iota(jnp.int32, (block, g), 1)).astype(
            jnp.float32
        )
        ub = jnp.dot(onehot, u_ref[...], preferred_element_type=jnp.float32)
        hcat = jnp.concatenate([x_b, agg, ub], axis=1)
        hid = jnp.dot(hcat, w1_ref[...], preferred_element_type=jnp.float32)
        hid = jnp.maximum(hid + b1_ref[...], 0.0)
        out = jnp.dot(hid, w2_ref[...], preferred_element_type=jnp.float32)
        out_ref[...] = out + b2_ref[...] + x_b

    return pl.pallas_call(
        body,
        grid=grid,
        in_specs=[
            pl.BlockSpec((block, d), lambda i: (i, 0)),
            pl.BlockSpec((_NC, block, de), lambda i: (0, i, 0)),
            pl.BlockSpec((block, 1), lambda i: (i, 0)),
            pl.BlockSpec((g, gf), lambda i: (0, 0)),
            pl.BlockSpec((d + de + gf, h), lambda i: (0, 0)),
            pl.BlockSpec((1, h), lambda i: (0, 0)),
            pl.BlockSpec((h, d), lambda i: (0, 0)),
            pl.BlockSpec((1, d), lambda i: (0, 0)),
        ],
        out_specs=pl.BlockSpec((block, d), lambda i: (i, 0)),
        out_shape=jax.ShapeDtypeStruct((n_nodes, d), jnp.float32),
        compiler_params=pltpu.CompilerParams(
            dimension_semantics=("parallel",),
        ),
    )


def kernel(x, edge_index, edge_attr, u, batch, W1, b1, W2, b2):
    n_nodes, d = x.shape
    n_edges, de = edge_attr.shape
    g, gf = u.shape
    h = W1.shape[1]

    a4 = edge_attr.T.reshape(de // 8, 8, n_edges // _CH, _CH).transpose(0, 2, 1, 3)
    ei3 = edge_index.reshape(2, n_edges // _CH, _CH).transpose(1, 0, 2)
    zeros = jnp.zeros((n_nodes, de), jnp.float32)
    partials = _sc_segment_sum(n_nodes, n_edges, de)(ei3, a4, zeros)

    mlp = _mlp(n_nodes, d, de, g, gf, h, block=1000)
    return mlp(
        x,
        partials,
        batch.reshape(n_nodes, 1),
        u,
        W1,
        b1.reshape(1, h),
        W2,
        b2.reshape(1, d),
    )


# speedup vs baseline: 7.4346x; 1.0337x over previous
import functools

import jax
import jax.numpy as jnp
from jax import lax
from jax.experimental import pallas as pl
from jax.experimental.pallas import tpu as pltpu
from jax.experimental.pallas import tpu_sc as plsc

_NC = 2
_NS = 16
_NW = _NC * _NS
_CH = 128


def _sc_segment_sum(n_nodes: int, n_edges: int, de: int):
    assert n_edges % _CH == 0
    n_chunks = n_edges // _CH
    base, rem = divmod(n_chunks, _NW)
    ss = 13
    assert base % ss == 0
    n_ss = base // ss
    w = ss * _CH
    rows_per_sub = (n_nodes // _NS) // 8 * 8
    tail_rows = n_nodes - rows_per_sub * _NS

    mesh = plsc.VectorSubcoreMesh(core_axis_name="c", subcore_axis_name="s")

    @functools.partial(
        pl.kernel,
        out_type=jax.ShapeDtypeStruct((_NC, n_nodes, de), jnp.float32),
        mesh=mesh,
        scratch_types=[
            pltpu.VMEM((2, w), jnp.int32),
            pltpu.VMEM((2, de, w), jnp.float32),
            pltpu.VMEM((2, w, de), jnp.float32),
            pltpu.VMEM_SHARED((n_nodes, de), jnp.float32),
            pltpu.SemaphoreType.DMA,
            pltpu.SemaphoreType.DMA,
            pltpu.SemaphoreType.DMA,
        ],
        compiler_params=pltpu.CompilerParams(
            use_tc_tiling_on_sc=False, needs_layout_passes=False
        ),
    )
    def scatter_kernel(
        ei3_hbm, a4_hbm, zero_hbm, out_hbm, idx_v, tbuf_v, rows_v, acc_sh,
        sem_i, sem_r, sem_s,
    ):
        c = lax.axis_index("c")
        s = lax.axis_index("s")
        wid = s * _NC + c

        zoff = s * rows_per_sub
        pltpu.sync_copy(
            zero_hbm.at[pl.ds(zoff, rows_per_sub), :],
            acc_sh.at[pl.ds(zoff, rows_per_sub), :],
        )
        if tail_rows:
            toff = rows_per_sub * _NS

            @pl.when(s == _NS - 1)
            def _zero_tail():
                pltpu.sync_copy(
                    zero_hbm.at[pl.ds(toff, tail_rows), :],
                    acc_sh.at[pl.ds(toff, tail_rows), :],
                )

        plsc.subcore_barrier()

        start = wid * base
        iota16 = lax.iota(jnp.int32, 16)

        def transpose_edges(buf, n_e):
            src2d = tbuf_v.at[buf]
            dst2d = rows_v.at[buf]

            @plsc.parallel_loop(0, n_e, 1, unroll=16, carry=jnp.zeros((16,), jnp.int32))
            def _tr(jj, vj):
                row = plsc.load_gather(src2d, [iota16, vj])
                plsc.store_scatter(dst2d, [vj, iota16], row)
                return vj + 1

        pending = {}

        def start_loads(t, buf):
            st = start + t * ss
            descs = []
            for k in range(ss):
                descs.append(
                    pltpu.async_copy(
                        ei3_hbm.at[st + k, 1], idx_v.at[buf, pl.ds(k * _CH, _CH)],
                        sem_i,
                    )
                )
                for tr in range(de // 8):
                    descs.append(
                        pltpu.async_copy(
                            a4_hbm.at[tr, st + k],
                            tbuf_v.at[buf, pl.ds(tr * 8, 8), pl.ds(k * _CH, _CH)],
                            sem_r,
                        )
                    )
            pending[buf] = descs

        start_loads(0, 0)
        for t in range(n_ss):
            b = t % 2
            if t + 1 < n_ss:
                start_loads(t + 1, 1 - b)
            for dsc in pending[b]:
                dsc.wait()
            transpose_edges(b, w)
            scats = [
                pltpu.async_copy(
                    rows_v.at[b, pl.ds(j * _CH, _CH), :],
                    acc_sh.at[idx_v.at[b, pl.ds(j * _CH, _CH)]],
                    sem_s,
                    add=True,
                )
                for j in range(ss)
            ]
            for sd in scats:
                sd.wait()

        if rem:

            @pl.when(wid < rem)
            def _extra_chunk():
                ec = _NW * base + wid
                pltpu.sync_copy(ei3_hbm.at[ec, 1], idx_v.at[0, pl.ds(0, _CH)])
                for tr in range(de // 8):
                    pltpu.sync_copy(
                        a4_hbm.at[tr, ec],
                        tbuf_v.at[0, pl.ds(tr * 8, 8), pl.ds(0, _CH)],
                    )
                transpose_edges(0, _CH)
                pltpu.sync_copy(
                    rows_v.at[0, pl.ds(0, _CH), :],
                    acc_sh.at[idx_v.at[0, pl.ds(0, _CH)]],
                    add=True,
                )

        plsc.subcore_barrier()

        pltpu.sync_copy(
            acc_sh.at[pl.ds(zoff, rows_per_sub), :],
            out_hbm.at[c, pl.ds(zoff, rows_per_sub), :],
        )
        if tail_rows:
            toff2 = rows_per_sub * _NS

            @pl.when(s == _NS - 1)
            def _write_tail():
                pltpu.sync_copy(
                    acc_sh.at[pl.ds(toff2, tail_rows), :],
                    out_hbm.at[c, pl.ds(toff2, tail_rows), :],
                )

    return scatter_kernel


def _mlp(n_nodes: int, d: int, de: int, g: int, gf: int, h: int, block: int):
    assert n_nodes % block == 0
    grid = (n_nodes // block,)

    def body(x_ref, pp_ref, batch_ref, u_ref, w1_ref, b1_ref, w2_ref, b2_ref, out_ref):
        x_b = x_ref[...]
        agg = pp_ref[0] + pp_ref[1]
        bcol = batch_ref[...]
        onehot = (bcol == lax.broadcasted_iota(jnp.int32, (block, g), 1)).astype(
            jnp.float32
        )
        ub = jnp.dot(onehot, u_ref[...], preferred_element_type=jnp.float32)
        hcat = jnp.concatenate([x_b, agg, ub], axis=1)
        hid = jnp.dot(hcat, w1_ref[...], preferred_element_type=jnp.float32)
        hid = jnp.maximum(hid + b1_ref[...], 0.0)
        out = jnp.dot(hid, w2_ref[...], preferred_element_type=jnp.float32)
        out_ref[...] = out + b2_ref[...] + x_b

    return pl.pallas_call(
        body,
        grid=grid,
        in_specs=[
            pl.BlockSpec((block, d), lambda i: (i, 0)),
            pl.BlockSpec((_NC, block, de), lambda i: (0, i, 0)),
            pl.BlockSpec((block, 1), lambda i: (i, 0)),
            pl.BlockSpec((g, gf), lambda i: (0, 0)),
            pl.BlockSpec((d + de + gf, h), lambda i: (0, 0)),
            pl.BlockSpec((1, h), lambda i: (0, 0)),
            pl.BlockSpec((h, d), lambda i: (0, 0)),
            pl.BlockSpec((1, d), lambda i: (0, 0)),
        ],
        out_specs=pl.BlockSpec((block, d), lambda i: (i, 0)),
        out_shape=jax.ShapeDtypeStruct((n_nodes, d), jnp.float32),
        compiler_params=pltpu.CompilerParams(
            dimension_semantics=("parallel",),
        ),
    )


def kernel(x, edge_index, edge_attr, u, batch, W1, b1, W2, b2):
    n_nodes, d = x.shape
    n_edges, de = edge_attr.shape
    g, gf = u.shape
    h = W1.shape[1]

    a4 = edge_attr.T.reshape(de // 8, 8, n_edges // _CH, _CH).transpose(0, 2, 1, 3)
    ei3 = edge_index.reshape(2, n_edges // _CH, _CH).transpose(1, 0, 2)
    zeros = jnp.zeros((n_nodes, de), jnp.float32)
    partials = _sc_segment_sum(n_nodes, n_edges, de)(ei3, a4, zeros)

    mlp = _mlp(n_nodes, d, de, g, gf, h, block=1000)
    return mlp(
        x,
        partials,
        batch.reshape(n_nodes, 1),
        u,
        W1,
        b1.reshape(1, h),
        W2,
        b2.reshape(1, d),
    )
